# baseline (speedup 1.0000x reference)
import functools

import jax
import jax.numpy as jnp
from jax import lax
from jax.experimental import pallas as pl
from jax.experimental.pallas import tpu as pltpu
from jax.experimental.pallas import tpu_sc as plsc

B = 16
L = 512
H = 16
V = 64

NC = 2
NS = 16
NW = NC * NS

ROWS = B * L
RPW = ROWS // NW
R = 4
CH = R * L
NB = CH // 128
NCHUNK = RPW // R
NPAIR = NCHUNK // 2


def _sc_body(tbl_hbm, data_hbm, out_hbm, tbl_sh, idx0, idx1, out0, out1,
             sem_i0, sem_i1, sem_g0, sem_g1, sem_o0, sem_o1):
    wid = lax.axis_index("s") * NC + lax.axis_index("c")
    sid = lax.axis_index("s")

    @pl.when(sid == 0)
    def _():
        pltpu.sync_copy(tbl_hbm, tbl_sh)

    plsc.subcore_barrier()

    idx_v = (idx0, idx1)
    out_v = (out0, out1)
    sem_i = (sem_i0, sem_i1)
    sem_g = (sem_g0, sem_g1)
    sem_o = (sem_o0, sem_o1)

    pltpu.async_copy(data_hbm.at[pl.ds(wid * (RPW * L // 128), NB)], idx0,
                     sem_i0)

    def process(c, slot):
        pltpu.make_async_copy(
            data_hbm.at[pl.ds(0, NB)], idx_v[slot], sem_i[slot]).wait()

        @pl.when(c + 1 < NCHUNK)
        def _():
            blk = wid * (RPW * L // 128) + (c + 1) * NB
            pltpu.async_copy(
                data_hbm.at[pl.ds(blk, NB)], idx_v[1 - slot],
                sem_i[1 - slot])

        @pl.when(c >= 2)
        def _():
            for h in range(H):
                pltpu.make_async_copy(
                    out_v[slot].at[pl.ds(h * CH, CH)],
                    out_hbm.at[pl.ds(0, CH)], sem_o[slot]).wait()

        def blk_body(j, carry):
            for h in range(H):
                pltpu.async_copy(
                    tbl_sh.at[h].at[idx_v[slot].at[j]],
                    out_v[slot].at[pl.ds(h * CH + j * 128, 128)],
                    sem_g[slot])
            return carry

        lax.fori_loop(0, NB, blk_body, 0)

        pltpu.make_async_copy(
            out_hbm.at[pl.ds(0, H * CH)], out_v[slot], sem_g[slot]).wait()

        r0 = wid * RPW + c * R
        b = r0 // L
        i0 = r0 - b * L
        base = (b * H * L + i0) * L
        for h in range(H):
            pltpu.async_copy(
                out_v[slot].at[pl.ds(h * CH, CH)],
                out_hbm.at[pl.ds(base + h * L * L, CH)], sem_o[slot])

    def pair_body(p, carry):
        process(2 * p, 0)
        process(2 * p + 1, 1)
        return carry

    lax.fori_loop(0, NPAIR, pair_body, 0)

    for slot in range(2):
        for h in range(H):
            pltpu.make_async_copy(
                out_v[slot].at[pl.ds(h * CH, CH)],
                out_hbm.at[pl.ds(0, CH)], sem_o[slot]).wait()


@jax.jit
def _hop_bias_sc(tbl, data_blk):
    mesh = plsc.VectorSubcoreMesh(core_axis_name="c", subcore_axis_name="s")
    run = pl.kernel(
        _sc_body,
        out_type=jax.ShapeDtypeStruct((B * H * L * L,), jnp.float32),
        mesh=mesh,
        scratch_types=[
            pltpu.MemorySpace.VMEM_SHARED((H, V), jnp.float32),
            pltpu.VMEM((NB, 128), jnp.int32),
            pltpu.VMEM((NB, 128), jnp.int32),
            pltpu.VMEM((H * CH,), jnp.float32),
            pltpu.VMEM((H * CH,), jnp.float32),
            pltpu.SemaphoreType.DMA,
            pltpu.SemaphoreType.DMA,
            pltpu.SemaphoreType.DMA,
            pltpu.SemaphoreType.DMA,
            pltpu.SemaphoreType.DMA,
            pltpu.SemaphoreType.DMA,
        ],
        compiler_params=pltpu.CompilerParams(
            needs_layout_passes=False, use_tc_tiling_on_sc=False),
    )
    return run(tbl, data_blk)


def kernel(data, hop_embeddings):
    data_blk = data.reshape(-1, 128).astype(jnp.int32)
    out_flat = _hop_bias_sc(hop_embeddings.T, data_blk)
    return out_flat.reshape(B, H, L, L)


# speedup vs baseline: 2.2446x; 2.2446x over previous
import functools

import jax
import jax.numpy as jnp
from jax import lax
from jax.experimental import pallas as pl
from jax.experimental.pallas import tpu as pltpu
from jax.experimental.pallas import tpu_sc as plsc

B = 16
L = 512
H = 16
V = 64

NC = 2
NS = 16
NW = NC * NS

ROWS = B * L
RPW = ROWS // NW
R = 4
CH = R * L
NCHUNK = RPW // R
NPAIR = NCHUNK // 2
G = CH // 16


def _sc_body(tbl_hbm, data_hbm, out_hbm, tbl_v, idx0, idx1, out0, out1,
             sem_i0, sem_i1, sem_o0, sem_o1):
    wid = lax.axis_index("s") * NC + lax.axis_index("c")
    pltpu.sync_copy(tbl_hbm, tbl_v)

    idx_v = (idx0, idx1)
    out_v = (out0, out1)
    sem_i = (sem_i0, sem_i1)
    sem_o = (sem_o0, sem_o1)

    pltpu.async_copy(data_hbm.at[pl.ds(wid * RPW * L, CH)], idx0, sem_i0)

    def process(c, slot):
        pltpu.make_async_copy(
            data_hbm.at[pl.ds(0, CH)], idx_v[slot], sem_i[slot]).wait()

        @pl.when(c + 1 < NCHUNK)
        def _():
            r0n = wid * RPW + (c + 1) * R
            pltpu.async_copy(
                data_hbm.at[pl.ds(r0n * L, CH)], idx_v[1 - slot],
                sem_i[1 - slot])

        @pl.when(c >= 2)
        def _():
            for h in range(H):
                pltpu.make_async_copy(
                    out_v[slot].at[h], out_hbm.at[pl.ds(0, CH)],
                    sem_o[slot]).wait()

        @plsc.parallel_loop(0, CH, step=16, unroll=2)
        def group_body(g16):
            idxv = idx_v[slot][pl.ds(g16, 16)]
            for h in range(H):
                out_v[slot][h, pl.ds(g16, 16)] = plsc.load_gather(
                    tbl_v.at[pl.ds(h * V, V)], [idxv])

        r0 = wid * RPW + c * R
        b = r0 // L
        i0 = r0 - b * L
        base = (b * H * L + i0) * L
        for h in range(H):
            pltpu.async_copy(
                out_v[slot].at[h], out_hbm.at[pl.ds(base + h * L * L, CH)],
                sem_o[slot])

    def pair_body(p, carry):
        process(2 * p, 0)
        process(2 * p + 1, 1)
        return carry

    lax.fori_loop(0, NPAIR, pair_body, 0)

    for slot in range(2):
        for h in range(H):
            pltpu.make_async_copy(
                out_v[slot].at[h], out_hbm.at[pl.ds(0, CH)],
                sem_o[slot]).wait()


@jax.jit
def _hop_bias_sc(tbl, data_flat):
    mesh = plsc.VectorSubcoreMesh(core_axis_name="c", subcore_axis_name="s")
    run = pl.kernel(
        _sc_body,
        out_type=jax.ShapeDtypeStruct((B * H * L * L,), jnp.float32),
        mesh=mesh,
        scratch_types=[
            pltpu.VMEM((V * H,), jnp.float32),
            pltpu.VMEM((CH,), jnp.int32),
            pltpu.VMEM((CH,), jnp.int32),
            pltpu.VMEM((H, CH), jnp.float32),
            pltpu.VMEM((H, CH), jnp.float32),
            pltpu.SemaphoreType.DMA,
            pltpu.SemaphoreType.DMA,
            pltpu.SemaphoreType.DMA,
            pltpu.SemaphoreType.DMA,
        ],
        compiler_params=pltpu.CompilerParams(
            needs_layout_passes=False, use_tc_tiling_on_sc=False),
    )
    return run(tbl, data_flat)


def kernel(data, hop_embeddings):
    data_flat = data.reshape(-1).astype(jnp.int32)
    out_flat = _hop_bias_sc(hop_embeddings.T.reshape(-1), data_flat)
    return out_flat.reshape(B, H, L, L)
